# baseline (speedup 1.0000x reference)
import functools

import jax
import jax.numpy as jnp
from jax import lax
from jax.experimental import pallas as pl
from jax.experimental.pallas import tpu as pltpu
from jax.experimental.pallas import tpu_sc as plsc

N = 10000
D = 128
E = 160000
B = 64
NT = 16
VROWS = E // B
GSZ = 6
RPT = 162
GN = RPT // GSZ
SROWS = RPT * NT
NP = 10240
OPT = NP // NT
ECH = 64
TAILRB = (N // ECH) * ECH
NTAIL = N - TAILRB
TAILS = N // OPT


def _prep_body(xi, xt, wi, ali, ari, wt, alt, art,
               h1, h2, er1, er2):
    ones16 = jnp.ones((1, 16), jnp.float32)
    hv1 = jnp.dot(xi[...], wi[...], preferred_element_type=jnp.float32)
    h1[:, :D] = hv1
    h1[:, D:] = jnp.dot(hv1, ali[...],
                        preferred_element_type=jnp.float32) * ones16
    er1[...] = jnp.dot(xt[...], jnp.dot(wi[...], ari[...]),
                       preferred_element_type=jnp.float32) * ones16
    hv2 = jnp.dot(xt[...], wt[...], preferred_element_type=jnp.float32)
    h2[:, :D] = hv2
    h2[:, D:] = jnp.dot(hv2, alt[...],
                        preferred_element_type=jnp.float32) * ones16
    er2[...] = jnp.dot(xi[...], jnp.dot(wt[...], art[...]),
                       preferred_element_type=jnp.float32) * ones16


DW = D + 16

_prep = pl.pallas_call(
    _prep_body,
    out_shape=[
        jax.ShapeDtypeStruct((N, DW), jnp.float32),
        jax.ShapeDtypeStruct((N, DW), jnp.float32),
        jax.ShapeDtypeStruct((N, 16), jnp.float32),
        jax.ShapeDtypeStruct((N, 16), jnp.float32),
    ],
)

_mesh = plsc.VectorSubcoreMesh(core_axis_name="c", subcore_axis_name="s")


@functools.partial(
    pl.kernel,
    out_type=[
        jax.ShapeDtypeStruct((N, D), jnp.float32),
        jax.ShapeDtypeStruct((N, D), jnp.float32),
    ],
    mesh=_mesh,
    compiler_params=pltpu.CompilerParams(use_tc_tiling_on_sc=False,
                                         needs_layout_passes=False),
    scratch_types=[
        pltpu.VMEM((2, 2, GSZ, B), jnp.int32),
        pltpu.VMEM((B, DW), jnp.float32),
        pltpu.VMEM((B, DW), jnp.float32),
        pltpu.VMEM((B, DW), jnp.float32),
        pltpu.VMEM((3, B, 16), jnp.float32),
        pltpu.VMEM((1, D), jnp.float32),
        pltpu.SemaphoreType.DMA,
        pltpu.SemaphoreType.DMA,
        pltpu.SemaphoreType.DMA,
        pltpu.SemaphoreType.DMA,
        pltpu.SemaphoreType.DMA,
        pltpu.SemaphoreType.DMA,
        pltpu.VMEM_SHARED((NP, DW), jnp.float32),
    ],
)
def _edge(h1, e1, sd1, f1, b1, h2, e2, sd2, f2, b2, o1, o2,
          idx_v, rows0, rows1, rows2, erg_v, bias_v,
          gsem0, gsem1, gsem2, ssem0, ssem1, ssem2, u_sh):
    c = lax.axis_index("c")
    s = lax.axis_index("s")
    z16 = jnp.zeros((16,), jnp.float32)
    rows = (rows0, rows1, rows2)
    gsems = (gsem0, gsem1, gsem2)
    ssems = (ssem0, ssem1, ssem2)

    def do_conv(a_h, er_h, sd_h, feat_h, b_h, out_h):
        base = s * OPT
        pltpu.sync_copy(b_h, bias_v)

        @plsc.parallel_loop(0, B, unroll=4)
        def _(r):
            for j in range(DW // 16):
                rows0[r, pl.ds(j * 16, 16)] = z16
        for k in range(OPT // B):
            pltpu.sync_copy(rows0, u_sh.at[pl.ds(base + k * B, B)])

        def valid(i):
            return (i < RPT) & (s * RPT + i < VROWS)

        def load_group(g, gb):
            g0 = jnp.minimum(s * RPT + g * GSZ, SROWS - GSZ)
            pltpu.sync_copy(sd_h.at[:, pl.ds(g0, GSZ)], idx_v.at[gb])

        def issue_gather(i, p, gbuf, off):
            @pl.when(valid(i))
            def _():
                pltpu.async_copy(a_h.at[idx_v.at[gbuf, 0, off]], rows[p],
                                 gsems[p])
                pltpu.async_copy(er_h.at[idx_v.at[gbuf, 1, off]],
                                 erg_v.at[p], gsems[p])

        def step(i, k, gb, gb2):
            p = k % 3
            rv = rows[p]

            @pl.when(valid(i))
            def _():
                pltpu.make_async_copy(a_h.at[pl.ds(0, B)], rv, gsems[p]).wait()
                pltpu.make_async_copy(er_h.at[pl.ds(0, B)], erg_v.at[p],
                                      gsems[p]).wait()

                @plsc.parallel_loop(0, B)
                def _(r):
                    e = rv[r, pl.ds(D, 16)] + erg_v[p, r, pl.ds(0, 16)]
                    e = jnp.where(e >= 0.0, e, 0.2 * e)
                    exv = jnp.exp(e)
                    for j in range(D // 16):
                        sl = pl.ds(j * 16, 16)
                        rv[r, sl] = rv[r, sl] * exv
                    rv[r, pl.ds(D, 16)] = exv
                pltpu.async_copy(rv, u_sh.at[idx_v.at[gb, 1, k]], ssems[p],
                                 add=True)

            @pl.when((i >= 1) & valid(i - 1))
            def _():
                q = (p + 2) % 3
                pltpu.make_async_copy(u_sh.at[pl.ds(0, B)], rows[q],
                                      ssems[q]).wait()
            if k < 4:
                issue_gather(i + 2, (p + 2) % 3, gb, k + 2)
            else:
                issue_gather(i + 2, (p + 2) % 3, gb2, k - 4)

        load_group(0, 0)
        issue_gather(0, 0, 0, 0)
        issue_gather(1, 1, 0, 1)
        plsc.subcore_barrier()

        def group(g, carry):
            gb = lax.rem(g, 2)
            gb2 = 1 - gb
            i0 = g * GSZ
            step(i0, 0, gb, gb2)
            load_group(g + 1, gb2)
            for k in range(1, GSZ):
                step(i0 + k, k, gb, gb2)
            return carry
        lax.fori_loop(0, GN, group, 0)
        @pl.when(valid(RPT - 1))
        def _():
            q = (RPT - 1) % 3
            pltpu.make_async_copy(u_sh.at[pl.ds(0, B)], rows[q],
                                  ssems[q]).wait()
        plsc.subcore_barrier()

        def echunk(rb, nr):
            pltpu.sync_copy(u_sh.at[pl.ds(rb, nr)], rows0.at[pl.ds(0, nr)])
            pltpu.sync_copy(feat_h.at[pl.ds(rb, nr)],
                            rows1.at[pl.ds(0, nr), pl.ds(0, D)])

            @plsc.parallel_loop(0, nr)
            def _(r):
                iv = 1.0 / (rows0[r, pl.ds(D, 16)] + 1e-9)
                for j in range(D // 16):
                    sl = pl.ds(j * 16, 16)
                    v = rows0[r, sl] * iv + rows1[r, sl] + bias_v[0, sl]
                    rows0[r, sl] = jnp.where(v > 0.0, v, jnp.exp(v) - 1.0)
            pltpu.sync_copy(rows0.at[pl.ds(0, nr), pl.ds(0, D)],
                            out_h.at[pl.ds(rb, nr)])

        def full_chunk(k, carry):
            rb = base + k * ECH

            @pl.when(rb + ECH <= N)
            def _():
                echunk(rb, ECH)
            return carry
        lax.fori_loop(0, OPT // ECH, full_chunk, 0)

        @pl.when(s == TAILS)
        def _():
            echunk(TAILRB, NTAIL)

    @pl.when(c == 0)
    def _():
        do_conv(h1, e1, sd1, f1, b1, o1)

    @pl.when(c == 1)
    def _():
        do_conv(h2, e2, sd2, f2, b2, o2)


def kernel(x_image, x_text, edge_index_image, edge_index_text,
           W_img, a_l_img, a_r_img, b_img, W_txt, a_l_txt, a_r_txt, b_txt):
    h1, h2, er1, er2 = _prep(
        x_image, x_text,
        W_img, a_l_img.reshape(D, 1), a_r_img.reshape(D, 1),
        W_txt, a_l_txt.reshape(D, 1), a_r_txt.reshape(D, 1),
    )
    pad = ((0, 0), (0, SROWS - VROWS), (0, 0))
    sd1 = jnp.pad(edge_index_image.astype(jnp.int32).reshape(2, VROWS, B),
                  pad)
    sd2 = jnp.pad(edge_index_text.astype(jnp.int32).reshape(2, VROWS, B),
                  pad)
    new_text, new_image = _edge(
        h1, er1, sd1, x_text, b_img.reshape(1, D),
        h2, er2, sd2, x_image, b_txt.reshape(1, D),
    )
    return (new_image, new_text)


# speedup vs baseline: 1.0406x; 1.0406x over previous
import functools

import jax
import jax.numpy as jnp
from jax import lax
from jax.experimental import pallas as pl
from jax.experimental.pallas import tpu as pltpu
from jax.experimental.pallas import tpu_sc as plsc

N = 10000
D = 128
E = 160000
B = 64
NT = 16
VROWS = E // B
GSZ = 6
RPT = 162
GN = RPT // GSZ
SROWS = RPT * NT
NP = 10240
OPT = NP // NT
ECH = 64
TAILRB = (N // ECH) * ECH
NTAIL = N - TAILRB
TAILS = N // OPT


def _prep_body(xi, xt, wi, ali, ari, wt, alt, art,
               h1, h2, el1, er1, el2, er2):
    ones16 = jnp.ones((1, 16), jnp.float32)
    h1[...] = jnp.dot(xi[...], wi[...], preferred_element_type=jnp.float32)
    el1[...] = jnp.dot(h1[...], ali[...],
                       preferred_element_type=jnp.float32) * ones16
    er1[...] = jnp.dot(xt[...], jnp.dot(wi[...], ari[...]),
                       preferred_element_type=jnp.float32) * ones16
    h2[...] = jnp.dot(xt[...], wt[...], preferred_element_type=jnp.float32)
    el2[...] = jnp.dot(h2[...], alt[...],
                       preferred_element_type=jnp.float32) * ones16
    er2[...] = jnp.dot(xi[...], jnp.dot(wt[...], art[...]),
                       preferred_element_type=jnp.float32) * ones16


_prep = pl.pallas_call(
    _prep_body,
    out_shape=[
        jax.ShapeDtypeStruct((N, D), jnp.float32),
        jax.ShapeDtypeStruct((N, D), jnp.float32),
        jax.ShapeDtypeStruct((N, 16), jnp.float32),
        jax.ShapeDtypeStruct((N, 16), jnp.float32),
        jax.ShapeDtypeStruct((N, 16), jnp.float32),
        jax.ShapeDtypeStruct((N, 16), jnp.float32),
    ],
)

_mesh = plsc.VectorSubcoreMesh(core_axis_name="c", subcore_axis_name="s")


@functools.partial(
    pl.kernel,
    out_type=[
        jax.ShapeDtypeStruct((N, D), jnp.float32),
        jax.ShapeDtypeStruct((N, D), jnp.float32),
    ],
    mesh=_mesh,
    compiler_params=pltpu.CompilerParams(use_tc_tiling_on_sc=False,
                                         needs_layout_passes=False),
    scratch_types=[
        pltpu.VMEM((2, 2, GSZ, B), jnp.int32),
        pltpu.VMEM((B, D), jnp.float32),
        pltpu.VMEM((B, D), jnp.float32),
        pltpu.VMEM((B, D), jnp.float32),
        pltpu.VMEM((3, B, 16), jnp.float32),
        pltpu.VMEM((3, B, 16), jnp.float32),
        pltpu.VMEM((3, B, 16), jnp.float32),
        pltpu.VMEM((1, D), jnp.float32),
        pltpu.SemaphoreType.DMA,
        pltpu.SemaphoreType.DMA,
        pltpu.SemaphoreType.DMA,
        pltpu.SemaphoreType.DMA,
        pltpu.SemaphoreType.DMA,
        pltpu.SemaphoreType.DMA,
        pltpu.VMEM_SHARED((NP, D), jnp.float32),
        pltpu.VMEM_SHARED((NP, 16), jnp.float32),
    ],
)
def _edge(h1, l1, e1, sd1, f1, b1, h2, l2, e2, sd2, f2, b2, o1, o2,
          idx_v, rows0, rows1, rows2, elg_v, erg_v, exw_v, bias_v,
          gsem0, gsem1, gsem2, ssem0, ssem1, ssem2, u_sh, s_sh):
    c = lax.axis_index("c")
    s = lax.axis_index("s")
    z16 = jnp.zeros((16,), jnp.float32)
    rows = (rows0, rows1, rows2)
    gsems = (gsem0, gsem1, gsem2)
    ssems = (ssem0, ssem1, ssem2)

    def do_conv(a_h, el_h, er_h, sd_h, feat_h, b_h, out_h):
        base = s * OPT
        pltpu.sync_copy(b_h, bias_v)

        @plsc.parallel_loop(0, B, unroll=4)
        def _(r):
            for j in range(D // 16):
                rows0[r, pl.ds(j * 16, 16)] = z16
            exw_v[0, r, pl.ds(0, 16)] = z16
        for k in range(OPT // B):
            pltpu.sync_copy(rows0, u_sh.at[pl.ds(base + k * B, B)])
            pltpu.sync_copy(exw_v.at[0], s_sh.at[pl.ds(base + k * B, B)])

        def valid(i):
            return (i < RPT) & (s * RPT + i < VROWS)

        def load_group(g, gb):
            g0 = jnp.minimum(s * RPT + g * GSZ, SROWS - GSZ)
            pltpu.sync_copy(sd_h.at[:, pl.ds(g0, GSZ)], idx_v.at[gb])

        def issue_gather(i, p, gbuf, off):
            @pl.when(valid(i))
            def _():
                pltpu.async_copy(a_h.at[idx_v.at[gbuf, 0, off]], rows[p],
                                 gsems[p])
                pltpu.async_copy(el_h.at[idx_v.at[gbuf, 0, off]],
                                 elg_v.at[p], gsems[p])
                pltpu.async_copy(er_h.at[idx_v.at[gbuf, 1, off]],
                                 erg_v.at[p], gsems[p])

        def step(i, k, gb, gb2):
            p = k % 3
            rv = rows[p]

            @pl.when(valid(i))
            def _():
                pltpu.make_async_copy(a_h.at[pl.ds(0, B)], rv, gsems[p]).wait()
                pltpu.make_async_copy(el_h.at[pl.ds(0, B)], elg_v.at[p],
                                      gsems[p]).wait()
                pltpu.make_async_copy(er_h.at[pl.ds(0, B)], erg_v.at[p],
                                      gsems[p]).wait()

                @plsc.parallel_loop(0, B)
                def _(r):
                    e = elg_v[p, r, pl.ds(0, 16)] + erg_v[p, r, pl.ds(0, 16)]
                    e = jnp.where(e >= 0.0, e, 0.2 * e)
                    exv = jnp.exp(e)
                    for j in range(D // 16):
                        sl = pl.ds(j * 16, 16)
                        rv[r, sl] = rv[r, sl] * exv
                    exw_v[p, r, pl.ds(0, 16)] = exv
                pltpu.async_copy(rv, u_sh.at[idx_v.at[gb, 1, k]], ssems[p],
                                 add=True)
                pltpu.async_copy(exw_v.at[p], s_sh.at[idx_v.at[gb, 1, k]],
                                 ssems[p], add=True)

            @pl.when((i >= 1) & valid(i - 1))
            def _():
                q = (p + 2) % 3
                pltpu.make_async_copy(a_h.at[pl.ds(0, B)], rows[q],
                                      ssems[q]).wait()
                pltpu.make_async_copy(el_h.at[pl.ds(0, B)], exw_v.at[q],
                                      ssems[q]).wait()
            if k < 4:
                issue_gather(i + 2, (p + 2) % 3, gb, k + 2)
            else:
                issue_gather(i + 2, (p + 2) % 3, gb2, k - 4)

        load_group(0, 0)
        issue_gather(0, 0, 0, 0)
        issue_gather(1, 1, 0, 1)
        plsc.subcore_barrier()

        def group(g, carry):
            gb = lax.rem(g, 2)
            gb2 = 1 - gb
            i0 = g * GSZ
            step(i0, 0, gb, gb2)
            load_group(g + 1, gb2)
            for k in range(1, GSZ):
                step(i0 + k, k, gb, gb2)
            return carry
        lax.fori_loop(0, GN, group, 0)
        @pl.when(valid(RPT - 1))
        def _():
            q = (RPT - 1) % 3
            pltpu.make_async_copy(a_h.at[pl.ds(0, B)], rows[q],
                                  ssems[q]).wait()
            pltpu.make_async_copy(el_h.at[pl.ds(0, B)], exw_v.at[q],
                                  ssems[q]).wait()
        plsc.subcore_barrier()

        def echunk(rb, nr):
            pltpu.sync_copy(u_sh.at[pl.ds(rb, nr)], rows0.at[pl.ds(0, nr)])
            pltpu.sync_copy(s_sh.at[pl.ds(rb, nr)],
                            exw_v.at[0, pl.ds(0, nr)])
            pltpu.sync_copy(feat_h.at[pl.ds(rb, nr)], rows1.at[pl.ds(0, nr)])

            @plsc.parallel_loop(0, nr)
            def _(r):
                iv = 1.0 / (exw_v[0, r, pl.ds(0, 16)] + 1e-9)
                for j in range(D // 16):
                    sl = pl.ds(j * 16, 16)
                    v = rows0[r, sl] * iv + rows1[r, sl] + bias_v[0, sl]
                    rows0[r, sl] = jnp.where(v > 0.0, v, jnp.exp(v) - 1.0)
            pltpu.sync_copy(rows0.at[pl.ds(0, nr)], out_h.at[pl.ds(rb, nr)])

        def full_chunk(k, carry):
            rb = base + k * ECH

            @pl.when(rb + ECH <= N)
            def _():
                echunk(rb, ECH)
            return carry
        lax.fori_loop(0, OPT // ECH, full_chunk, 0)

        @pl.when(s == TAILS)
        def _():
            echunk(TAILRB, NTAIL)

    @pl.when(c == 0)
    def _():
        do_conv(h1, l1, e1, sd1, f1, b1, o1)

    @pl.when(c == 1)
    def _():
        do_conv(h2, l2, e2, sd2, f2, b2, o2)


def kernel(x_image, x_text, edge_index_image, edge_index_text,
           W_img, a_l_img, a_r_img, b_img, W_txt, a_l_txt, a_r_txt, b_txt):
    h1, h2, el1, er1, el2, er2 = _prep(
        x_image, x_text,
        W_img, a_l_img.reshape(D, 1), a_r_img.reshape(D, 1),
        W_txt, a_l_txt.reshape(D, 1), a_r_txt.reshape(D, 1),
    )
    pad = ((0, 0), (0, SROWS - VROWS), (0, 0))
    sd1 = jnp.pad(edge_index_image.astype(jnp.int32).reshape(2, VROWS, B),
                  pad)
    sd2 = jnp.pad(edge_index_text.astype(jnp.int32).reshape(2, VROWS, B),
                  pad)
    new_text, new_image = _edge(
        h1, el1, er1, sd1, x_text, b_img.reshape(1, D),
        h2, el2, er2, sd2, x_image, b_txt.reshape(1, D),
    )
    return (new_image, new_text)
